# Initial kernel scaffold; baseline (speedup 1.0000x reference)
import functools

import jax
import jax.numpy as jnp
from jax import lax
from jax.experimental import pallas as pl
from jax.experimental.pallas import tpu as pltpu
from jax.experimental.pallas import tpu_sc as plsc

B = 16384
D = 64
NEG = 20
NC = 2
NS = 16
NW = NC * NS
BPW = B // NW
CHUNK = 64
NCHUNK = BPW // CHUNK
IDX_DMA = 128
SCORE_W = 32


@functools.partial(
    pl.kernel,
    out_type=jax.ShapeDtypeStruct((B, SCORE_W), jnp.float32),
    mesh=plsc.VectorSubcoreMesh(
        core_axis_name="c", subcore_axis_name="s", num_cores=NC, num_subcores=NS
    ),
    scratch_types=[
        pltpu.VMEM((BPW,), jnp.int32),
        pltpu.VMEM((BPW,), jnp.int32),
        pltpu.VMEM((BPW * NEG,), jnp.int32),
        pltpu.VMEM((CHUNK, D), jnp.float32),
        pltpu.VMEM((CHUNK, D), jnp.float32),
        pltpu.VMEM((CHUNK * NEG, D), jnp.float32),
        pltpu.VMEM((CHUNK, SCORE_W), jnp.float32),
        pltpu.SemaphoreType.DMA,
    ],
)
def _sc_scores(v_emb, u_emb, v_h, up_h, un_h, out,
               v_idx, up_idx, un_idx, v_rows, up_rows, un_rows, scores, sem):
    wid = lax.axis_index("s") * NC + lax.axis_index("c")
    base = wid * BPW

    pltpu.sync_copy(v_h.at[pl.ds(base, BPW)], v_idx)
    pltpu.sync_copy(up_h.at[pl.ds(base, BPW)], up_idx)
    pltpu.sync_copy(un_h.at[pl.ds(base * NEG, BPW * NEG)], un_idx)

    for c in range(NCHUNK):
        cb = c * CHUNK
        cps = [
            pltpu.async_copy(v_emb.at[v_idx.at[pl.ds(cb, CHUNK)]], v_rows, sem),
            pltpu.async_copy(u_emb.at[up_idx.at[pl.ds(cb, CHUNK)]], up_rows, sem),
        ]
        for k in range(CHUNK * NEG // IDX_DMA):
            cps.append(pltpu.async_copy(
                u_emb.at[un_idx.at[pl.ds(cb * NEG + k * IDX_DMA, IDX_DMA)]],
                un_rows.at[pl.ds(k * IDX_DMA, IDX_DMA)], sem))
        for cp in cps:
            cp.wait()

        def elem(e, carry):
            vr = [v_rows[e, pl.ds(16 * j, 16)] for j in range(4)]
            scores[e, pl.ds(16, 16)] = jnp.zeros((16,), jnp.float32)
            up = [up_rows[e, pl.ds(16 * j, 16)] for j in range(4)]
            acc = vr[0] * up[0]
            for j in range(1, 4):
                acc = acc + vr[j] * up[j]
            scores[e, 0] = jnp.sum(acc)
            nv = [-x for x in vr]
            for n in range(NEG):
                r = e * NEG + n
                un = [un_rows[r, pl.ds(16 * j, 16)] for j in range(4)]
                a = nv[0] * un[0]
                for j in range(1, 4):
                    a = a + nv[j] * un[j]
                scores[e, n + 1] = jnp.sum(a)
            return carry

        lax.fori_loop(0, CHUNK, elem, 0)
        pltpu.sync_copy(scores, out.at[pl.ds(base + cb, CHUNK)])


def _tc_reduce(scores):
    nblk = 16

    def body(s_ref, o_ref):
        i = pl.program_id(0)

        @pl.when(i == 0)
        def _init():
            o_ref[0, 0] = jnp.float32(0.0)

        x = s_ref[...]
        col = lax.broadcasted_iota(jnp.int32, x.shape, 1)
        ls = jnp.minimum(x, 0.0) - jnp.log1p(jnp.exp(-jnp.abs(x)))
        ls = jnp.where(col < NEG + 1, ls, 0.0)
        o_ref[0, 0] += jnp.sum(ls)

    return pl.pallas_call(
        body,
        grid=(nblk,),
        in_specs=[pl.BlockSpec((B // nblk, SCORE_W), lambda i: (i, 0))],
        out_specs=pl.BlockSpec(memory_space=pltpu.SMEM),
        out_shape=jax.ShapeDtypeStruct((1, 1), jnp.float32),
    )(scores)


def kernel(v_emb, u_emb, v, u_pos, u_neg):
    v = v.astype(jnp.int32)
    u_pos = u_pos.astype(jnp.int32)
    u_neg_flat = u_neg.astype(jnp.int32).reshape(B * NEG)
    scores = _sc_scores(v_emb, u_emb, v, u_pos, u_neg_flat)
    tot = _tc_reduce(scores)
    return -tot[0, 0]


# speedup vs baseline: 4.6388x; 4.6388x over previous
import functools

import jax
import jax.numpy as jnp
from jax import lax
from jax.experimental import pallas as pl
from jax.experimental.pallas import tpu as pltpu
from jax.experimental.pallas import tpu_sc as plsc

B = 16384
D = 64
NEG = 20
NC = 2
NS = 16
NW = NC * NS
BPW = B // NW
CHUNK = 64
NCHUNK = BPW // CHUNK
IDX_DMA = 128
SCORE_W = 32


def _hsum(x, perms):
    for p in perms:
        x = x + x.at[p].get(mode="promise_in_bounds")
    return x


@functools.partial(
    pl.kernel,
    out_type=jax.ShapeDtypeStruct((B * SCORE_W,), jnp.float32),
    mesh=plsc.VectorSubcoreMesh(
        core_axis_name="c", subcore_axis_name="s", num_cores=NC, num_subcores=NS
    ),
    scratch_types=[
        pltpu.VMEM((BPW,), jnp.int32),
        pltpu.VMEM((BPW,), jnp.int32),
        pltpu.VMEM((BPW * NEG,), jnp.int32),
        pltpu.VMEM((CHUNK, D), jnp.float32),
        pltpu.VMEM((CHUNK, D), jnp.float32),
        pltpu.VMEM((CHUNK * NEG, D), jnp.float32),
        pltpu.VMEM((CHUNK * SCORE_W,), jnp.float32),
        pltpu.SemaphoreType.DMA,
    ],
    compiler_params=pltpu.CompilerParams(
        needs_layout_passes=False, use_tc_tiling_on_sc=False
    ),
)
def _sc_scores(v_emb, u_emb, v_h, up_h, un_h, out,
               v_idx, up_idx, un_idx, v_rows, up_rows, un_rows, scores, sem):
    wid = lax.axis_index("s") * NC + lax.axis_index("c")
    base = wid * BPW

    pltpu.sync_copy(v_h.at[pl.ds(base, BPW)], v_idx)
    pltpu.sync_copy(up_h.at[pl.ds(base, BPW)], up_idx)
    pltpu.sync_copy(un_h.at[pl.ds(base * NEG, BPW * NEG)], un_idx)

    lanes = lax.iota(jnp.int32, 16)
    perms = [lanes ^ k for k in (8, 4, 2, 1)]
    lane0 = lanes == 0
    zeros16 = jnp.zeros((16,), jnp.float32)

    for c in range(NCHUNK):
        cb = c * CHUNK
        cps = [
            pltpu.async_copy(v_emb.at[v_idx.at[pl.ds(cb, CHUNK)]], v_rows, sem),
            pltpu.async_copy(u_emb.at[up_idx.at[pl.ds(cb, CHUNK)]], up_rows, sem),
        ]
        for k in range(CHUNK * NEG // IDX_DMA):
            cps.append(pltpu.async_copy(
                u_emb.at[un_idx.at[pl.ds(cb * NEG + k * IDX_DMA, IDX_DMA)]],
                un_rows.at[pl.ds(k * IDX_DMA, IDX_DMA)], sem))

        def zpad(e, carry):
            scores[pl.ds(e * SCORE_W + 16, 16)] = zeros16
            return carry

        lax.fori_loop(0, CHUNK, zpad, 0)
        for cp in cps:
            cp.wait()

        def elem(e, carry):
            obase = e * SCORE_W
            vr = [v_rows[e, pl.ds(16 * j, 16)] for j in range(4)]
            up = [up_rows[e, pl.ds(16 * j, 16)] for j in range(4)]
            acc = vr[0] * up[0]
            for j in range(1, 4):
                acc = acc + vr[j] * up[j]
            s = _hsum(acc, perms)
            plsc.store_scatter(scores, [jnp.broadcast_to(obase, (16,))], s,
                               mask=lane0)
            nv = [-x for x in vr]
            for n in range(NEG):
                r = e * NEG + n
                un = [un_rows[r, pl.ds(16 * j, 16)] for j in range(4)]
                a = nv[0] * un[0]
                for j in range(1, 4):
                    a = a + nv[j] * un[j]
                s = _hsum(a, perms)
                plsc.store_scatter(scores,
                                   [jnp.broadcast_to(obase + n + 1, (16,))], s,
                                   mask=lane0)
            return carry

        lax.fori_loop(0, CHUNK, elem, 0)
        pltpu.sync_copy(scores,
                        out.at[pl.ds((base + cb) * SCORE_W, CHUNK * SCORE_W)])


def _tc_reduce(scores):
    nblk = 16

    def body(s_ref, o_ref):
        i = pl.program_id(0)

        @pl.when(i == 0)
        def _init():
            o_ref[0, 0] = jnp.float32(0.0)

        x = s_ref[...]
        col = lax.broadcasted_iota(jnp.int32, x.shape, 1)
        ls = jnp.minimum(x, 0.0) - jnp.log1p(jnp.exp(-jnp.abs(x)))
        ls = jnp.where(col < NEG + 1, ls, 0.0)
        o_ref[0, 0] += jnp.sum(ls)

    return pl.pallas_call(
        body,
        grid=(nblk,),
        in_specs=[pl.BlockSpec((B // nblk, SCORE_W), lambda i: (i, 0))],
        out_specs=pl.BlockSpec(memory_space=pltpu.SMEM),
        out_shape=jax.ShapeDtypeStruct((1, 1), jnp.float32),
    )(scores)


def kernel(v_emb, u_emb, v, u_pos, u_neg):
    v = v.astype(jnp.int32)
    u_pos = u_pos.astype(jnp.int32)
    u_neg_flat = u_neg.astype(jnp.int32).reshape(B * NEG)
    scores = _sc_scores(v_emb, u_emb, v, u_pos, u_neg_flat)
    tot = _tc_reduce(scores.reshape(B, SCORE_W))
    return -tot[0, 0]
